# baseline (speedup 1.0000x reference)
import jax
import jax.numpy as jnp
from jax import lax
from jax.experimental import pallas as pl
from jax.experimental.pallas import tpu as pltpu
from jax.experimental.pallas import tpu_sc as plsc

_STRIDE_SHIFT = 3
_BATCH = 2
_NX, _NH, _NZ = 200, 4, 176
_ROWS_PER_BATCH = _NX * _NZ * _NH
_TOTAL_ROWS = _BATCH * _ROWS_PER_BATCH
_C = 128

_NCORES = 2
_NTILES = 16
_L = 16
_R = 12800
_NSHARDS = _TOTAL_ROWS // _R
_ACC_ROWS = _R + _L
_FROWS = _R // _NTILES

_NPAD = 100352
_PTS_PER_TILE = _NPAD // _NTILES
_VREGS_PER_TILE = _PTS_PER_TILE // _L
_BIDX = 64
_LIST_CAP = 6400

_CHUNKS = ((0, 12), (12, 22))


def _make_body(q0, q1, first):
    local_passes = (q1 - q0) // _NCORES

    def body(coords_t, feats, out, r_hbm, acc, r_v, loc_flat, pid_flat,
             pid_row0, loc_row0, pid_row1, loc_row1, rows_v, sem, sem2):
        core = lax.axis_index("c")
        tile = lax.axis_index("s")
        pbase = tile * _PTS_PER_TILE
        lane = lax.iota(jnp.int32, _L)

        if first:
            stage = pid_flat

            def _accum_component(row, fn, is_first):
                pltpu.sync_copy(coords_t.at[row, pl.ds(pbase, _PTS_PER_TILE)],
                                stage.at[pl.ds(0, _PTS_PER_TILE)])

                def step(i, _):
                    off = i * _L
                    v = fn(stage[pl.ds(off, _L)])
                    r_v[pl.ds(off, _L)] = (
                        v if is_first else r_v[pl.ds(off, _L)] + v)
                    return 0
                lax.fori_loop(0, _VREGS_PER_TILE, step, 0)

            _accum_component(
                3, lambda cb: cb * _ROWS_PER_BATCH, True)
            _accum_component(
                0, lambda cx: lax.shift_right_logical(cx, _STRIDE_SHIFT)
                * (_NZ * _NH), False)
            _accum_component(
                2, lambda cz: lax.shift_right_logical(cz, _STRIDE_SHIFT) * _NH,
                False)
            _accum_component(
                1, lambda ch: jnp.clip(
                    lax.shift_right_logical(ch, _STRIDE_SHIFT), 0, _NH - 1),
                False)
            pltpu.sync_copy(r_v, r_hbm.at[pl.ds(pbase, _PTS_PER_TILE)])
        else:
            pltpu.sync_copy(coords_t.at[pl.ds(pbase, _PTS_PER_TILE)], r_v)

        def _shard_base(p2):
            return (q0 + p2 * _NCORES + core) * _R

        def _compact(base):
            def step(i, ptr):
                off = i * _L
                r = r_v[pl.ds(off, _L)]
                loc = r - base
                mask = (loc >= 0) & (loc < _R)
                mi = mask.astype(jnp.int32)
                cum = plsc.cumsum(mi)
                pos = ptr + cum - 1
                pid = pbase + off + lane
                plsc.store_scatter(loc_flat, [pos], loc, mask=mask)
                plsc.store_scatter(pid_flat, [pos], pid, mask=mask)
                return ptr + jnp.sum(mi)
            m = lax.fori_loop(0, _VREGS_PER_TILE, step, jnp.int32(0))
            for k in range(_BIDX // _L + 1):
                loc_flat[pl.ds(m + k * _L, _L)] = _R + lane
                pid_flat[pl.ds(m + k * _L, _L)] = lane
            return m

        m0 = _compact(_shard_base(0))

        def one_pass(p2, m_cur):
            base = _shard_base(p2)

            zero16 = jnp.zeros((_L,), jnp.float32)

            def zrow(i, _):
                for c in range(_C // _L):
                    rows_v[i, pl.ds(c * _L, _L)] = zero16
                return 0
            lax.fori_loop(0, _BIDX, zrow, 0)

            zbase = tile * _FROWS
            zdescs = []
            for k in range(_FROWS // _BIDX):
                zdescs.append(pltpu.async_copy(
                    rows_v, acc.at[pl.ds(zbase + k * _BIDX, _BIDX)], sem))
            rem = _FROWS % _BIDX
            if rem:
                zdescs.append(pltpu.async_copy(
                    rows_v.at[pl.ds(0, rem)],
                    acc.at[pl.ds(zbase + _FROWS - rem, rem)], sem))
            for d in zdescs:
                d.wait()

            plsc.subcore_barrier()

            npair = (m_cur + _BIDX - 1) // _BIDX
            half = _BIDX // 2

            def one_pair(jj, _):
                fb0 = jj * _BIDX
                fb1 = fb0 + half
                for b in range(half // _L):
                    pid_row0[pl.ds(b * _L, _L)] = (
                        pid_flat[pl.ds(fb0 + b * _L, _L)])
                    loc_row0[pl.ds(b * _L, _L)] = (
                        loc_flat[pl.ds(fb0 + b * _L, _L)])
                g0 = pltpu.async_copy(
                    feats.at[pid_row0], rows_v.at[pl.ds(0, half)], sem)
                for b in range(half // _L):
                    pid_row1[pl.ds(b * _L, _L)] = (
                        pid_flat[pl.ds(fb1 + b * _L, _L)])
                    loc_row1[pl.ds(b * _L, _L)] = (
                        loc_flat[pl.ds(fb1 + b * _L, _L)])
                g1 = pltpu.async_copy(
                    feats.at[pid_row1], rows_v.at[pl.ds(half, half)], sem2)
                g0.wait()
                pltpu.sync_copy(rows_v.at[pl.ds(0, half)],
                                acc.at[loc_row0], add=True)
                g1.wait()
                pltpu.sync_copy(rows_v.at[pl.ds(half, half)],
                                acc.at[loc_row1], add=True)
                return 0
            lax.fori_loop(0, npair, one_pair, 0)

            plsc.subcore_barrier()

            fbase = tile * _FROWS
            fdesc = pltpu.async_copy(
                acc.at[pl.ds(fbase, _FROWS)],
                out.at[pl.ds(base - q0 * _R + fbase, _FROWS)], sem)
            m_next = _compact(_shard_base(p2 + 1))
            fdesc.wait()
            return m_next

        lax.fori_loop(0, local_passes, one_pass, m0)

    return body


def _make_call(q0, q1, first):
    nrows = (q1 - q0) * _R
    mesh = plsc.VectorSubcoreMesh(core_axis_name="c", subcore_axis_name="s")
    out_types = [jax.ShapeDtypeStruct((nrows, _C), jnp.float32)]
    if first:
        out_types.append(jax.ShapeDtypeStruct((_NPAD,), jnp.int32))

    body = _make_body(q0, q1, first)
    if first:
        def wrapped(coords_t, feats, out, r_out, *scratch):
            body(coords_t, feats, out, r_out, *scratch)
    else:
        def wrapped(r_hbm, feats, out, *scratch):
            body(r_hbm, feats, out, None, *scratch)

    return pl.kernel(
        wrapped,
        out_type=out_types if first else out_types[0],
        mesh=mesh,
        compiler_params=pltpu.CompilerParams(needs_layout_passes=False),
        scratch_types=[
            pltpu.VMEM_SHARED((_ACC_ROWS, _C), jnp.float32),
            pltpu.VMEM((_PTS_PER_TILE,), jnp.int32),
            pltpu.VMEM((_LIST_CAP,), jnp.int32),
            pltpu.VMEM((_LIST_CAP,), jnp.int32),
            pltpu.VMEM((_BIDX // 2,), jnp.int32),
            pltpu.VMEM((_BIDX // 2,), jnp.int32),
            pltpu.VMEM((_BIDX // 2,), jnp.int32),
            pltpu.VMEM((_BIDX // 2,), jnp.int32),
            pltpu.VMEM((_BIDX, _C), jnp.float32),
            pltpu.SemaphoreType.DMA,
            pltpu.SemaphoreType.DMA,
        ],
    )


def _tc_transpose_chunk(prev, table, q0, q1):
    ns = q1 - q0
    spb = _NSHARDS // _BATCH
    table2 = table.reshape(ns * (_R // _NH), _NH * _C)

    in_specs = [pl.BlockSpec((_R // _NH, _NH * _C), lambda s: (s, 0))]
    args = [table2]
    aliases = {}
    if prev is not None:
        in_specs = [pl.BlockSpec(memory_space=pl.ANY)] + in_specs
        args = [prev] + args
        aliases = {0: 0}

    def body(*refs):
        i_ref, o_ref = refs[-2], refs[-1]
        o_ref[0] = i_ref[...].T

    return pl.pallas_call(
        body,
        grid=(ns,),
        in_specs=in_specs,
        out_specs=pl.BlockSpec(
            (1, _NH * _C, _R // _NH),
            lambda s: ((q0 + s) // spb, 0, (q0 + s) % spb)),
        out_shape=jax.ShapeDtypeStruct(
            (_BATCH, _NH * _C, _NX * _NZ), jnp.float32),
        input_output_aliases=aliases,
    )(*args)


def kernel(coords, feats):
    n = coords.shape[0]
    pad = jnp.zeros((_NPAD - n, 4), jnp.int32).at[:, 3].set(_BATCH)
    coords_t = jnp.concatenate([coords.astype(jnp.int32), pad], axis=0).T

    tables = []
    r_hbm = None
    for ci, (q0, q1) in enumerate(_CHUNKS):
        call = _make_call(q0, q1, ci == 0)
        if ci == 0:
            table, r_hbm = call(coords_t, feats)
        else:
            table = call(r_hbm, feats)
        tables.append(table)

    out = None
    for (q0, q1), table in zip(_CHUNKS, tables):
        out = _tc_transpose_chunk(out, table, q0, q1)
    return out.reshape(_BATCH, _NH * _C, _NX, _NZ)


# speedup vs baseline: 1.4002x; 1.4002x over previous
import jax
import jax.numpy as jnp
from jax import lax
from jax.experimental import pallas as pl
from jax.experimental.pallas import tpu as pltpu
from jax.experimental.pallas import tpu_sc as plsc

_STRIDE_SHIFT = 3
_BATCH = 2
_NX, _NH, _NZ = 200, 4, 176
_ROWS_PER_BATCH = _NX * _NZ * _NH
_TOTAL_ROWS = _BATCH * _ROWS_PER_BATCH
_C = 128

_NCORES = 2
_NTILES = 16
_L = 16
_R = 12800
_NSHARDS = _TOTAL_ROWS // _R
_ACC_ROWS = _R + _L
_FROWS = _R // _NTILES

_NPAD = 100352
_PTS_PER_TILE = _NPAD // _NTILES
_VREGS_PER_TILE = _PTS_PER_TILE // _L
_BIDX = 64
_LIST_CAP = 6400

_CHUNKS = ((0, 22),)


def _make_body(q0, q1, first):
    local_passes = (q1 - q0) // _NCORES

    def body(coords_t, feats, out, r_hbm, acc, r_v, loc_flat, pid_flat,
             pid_row0, loc_row0, pid_row1, loc_row1, rows_v, sem, sem2):
        core = lax.axis_index("c")
        tile = lax.axis_index("s")
        pbase = tile * _PTS_PER_TILE
        lane = lax.iota(jnp.int32, _L)

        if first:
            stage = pid_flat

            def _accum_component(row, fn, is_first):
                pltpu.sync_copy(coords_t.at[row, pl.ds(pbase, _PTS_PER_TILE)],
                                stage.at[pl.ds(0, _PTS_PER_TILE)])

                def step(i, _):
                    off = i * _L
                    v = fn(stage[pl.ds(off, _L)])
                    r_v[pl.ds(off, _L)] = (
                        v if is_first else r_v[pl.ds(off, _L)] + v)
                    return 0
                lax.fori_loop(0, _VREGS_PER_TILE, step, 0)

            _accum_component(
                3, lambda cb: cb * _ROWS_PER_BATCH, True)
            _accum_component(
                0, lambda cx: lax.shift_right_logical(cx, _STRIDE_SHIFT)
                * (_NZ * _NH), False)
            _accum_component(
                2, lambda cz: lax.shift_right_logical(cz, _STRIDE_SHIFT) * _NH,
                False)
            _accum_component(
                1, lambda ch: jnp.clip(
                    lax.shift_right_logical(ch, _STRIDE_SHIFT), 0, _NH - 1),
                False)
            pltpu.sync_copy(r_v, r_hbm.at[pl.ds(pbase, _PTS_PER_TILE)])
        else:
            pltpu.sync_copy(coords_t.at[pl.ds(pbase, _PTS_PER_TILE)], r_v)

        def _shard_base(p2):
            return (q0 + p2 * _NCORES + core) * _R

        def _compact(base):
            def step(i, ptr):
                off = i * _L
                r = r_v[pl.ds(off, _L)]
                loc = r - base
                mask = (loc >= 0) & (loc < _R)
                mi = mask.astype(jnp.int32)
                cum = plsc.cumsum(mi)
                pos = ptr + cum - 1
                pid = pbase + off + lane
                plsc.store_scatter(loc_flat, [pos], loc, mask=mask)
                plsc.store_scatter(pid_flat, [pos], pid, mask=mask)
                return ptr + cum[15]
            m = lax.fori_loop(0, _VREGS_PER_TILE, step, jnp.int32(0))
            for k in range(_BIDX // _L + 1):
                loc_flat[pl.ds(m + k * _L, _L)] = _R + lane
                pid_flat[pl.ds(m + k * _L, _L)] = lane
            return m

        m0 = _compact(_shard_base(0))

        def one_pass(p2, m_cur):
            base = _shard_base(p2)

            zero16 = jnp.zeros((_L,), jnp.float32)

            def zrow(i, _):
                for c in range(_C // _L):
                    rows_v[i, pl.ds(c * _L, _L)] = zero16
                return 0
            lax.fori_loop(0, _BIDX, zrow, 0)

            zbase = tile * _FROWS
            zdescs = []
            for k in range(_FROWS // _BIDX):
                zdescs.append(pltpu.async_copy(
                    rows_v, acc.at[pl.ds(zbase + k * _BIDX, _BIDX)], sem))
            rem = _FROWS % _BIDX
            if rem:
                zdescs.append(pltpu.async_copy(
                    rows_v.at[pl.ds(0, rem)],
                    acc.at[pl.ds(zbase + _FROWS - rem, rem)], sem))
            for d in zdescs:
                d.wait()

            plsc.subcore_barrier()

            npair = (m_cur + _BIDX - 1) // _BIDX
            half = _BIDX // 2

            def one_pair(jj, _):
                fb0 = jj * _BIDX
                fb1 = fb0 + half
                for b in range(half // _L):
                    pid_row0[pl.ds(b * _L, _L)] = (
                        pid_flat[pl.ds(fb0 + b * _L, _L)])
                    loc_row0[pl.ds(b * _L, _L)] = (
                        loc_flat[pl.ds(fb0 + b * _L, _L)])
                g0 = pltpu.async_copy(
                    feats.at[pid_row0], rows_v.at[pl.ds(0, half)], sem)
                for b in range(half // _L):
                    pid_row1[pl.ds(b * _L, _L)] = (
                        pid_flat[pl.ds(fb1 + b * _L, _L)])
                    loc_row1[pl.ds(b * _L, _L)] = (
                        loc_flat[pl.ds(fb1 + b * _L, _L)])
                g1 = pltpu.async_copy(
                    feats.at[pid_row1], rows_v.at[pl.ds(half, half)], sem2)
                g0.wait()
                pltpu.sync_copy(rows_v.at[pl.ds(0, half)],
                                acc.at[loc_row0], add=True)
                g1.wait()
                pltpu.sync_copy(rows_v.at[pl.ds(half, half)],
                                acc.at[loc_row1], add=True)
                return 0
            lax.fori_loop(0, npair, one_pair, 0)

            plsc.subcore_barrier()

            fbase = tile * _FROWS
            fdesc = pltpu.async_copy(
                acc.at[pl.ds(fbase, _FROWS)],
                out.at[pl.ds(base - q0 * _R + fbase, _FROWS)], sem)
            m_next = _compact(_shard_base(p2 + 1))
            fdesc.wait()
            return m_next

        lax.fori_loop(0, local_passes, one_pass, m0)

    return body


def _make_call(q0, q1, first):
    nrows = (q1 - q0) * _R
    mesh = plsc.VectorSubcoreMesh(core_axis_name="c", subcore_axis_name="s")
    out_types = [jax.ShapeDtypeStruct((nrows, _C), jnp.float32)]
    if first:
        out_types.append(jax.ShapeDtypeStruct((_NPAD,), jnp.int32))

    body = _make_body(q0, q1, first)
    if first:
        def wrapped(coords_t, feats, out, r_out, *scratch):
            body(coords_t, feats, out, r_out, *scratch)
    else:
        def wrapped(r_hbm, feats, out, *scratch):
            body(r_hbm, feats, out, None, *scratch)

    return pl.kernel(
        wrapped,
        out_type=out_types if first else out_types[0],
        mesh=mesh,
        compiler_params=pltpu.CompilerParams(needs_layout_passes=False),
        scratch_types=[
            pltpu.VMEM_SHARED((_ACC_ROWS, _C), jnp.float32),
            pltpu.VMEM((_PTS_PER_TILE,), jnp.int32),
            pltpu.VMEM((_LIST_CAP,), jnp.int32),
            pltpu.VMEM((_LIST_CAP,), jnp.int32),
            pltpu.VMEM((_BIDX // 2,), jnp.int32),
            pltpu.VMEM((_BIDX // 2,), jnp.int32),
            pltpu.VMEM((_BIDX // 2,), jnp.int32),
            pltpu.VMEM((_BIDX // 2,), jnp.int32),
            pltpu.VMEM((_BIDX, _C), jnp.float32),
            pltpu.SemaphoreType.DMA,
            pltpu.SemaphoreType.DMA,
        ],
    )


def _tc_transpose_chunk(prev, table, q0, q1):
    ns = q1 - q0
    spb = _NSHARDS // _BATCH
    table2 = table.reshape(ns * (_R // _NH), _NH * _C)

    in_specs = [pl.BlockSpec((_R // _NH, _NH * _C), lambda s: (s, 0))]
    args = [table2]
    aliases = {}
    if prev is not None:
        in_specs = [pl.BlockSpec(memory_space=pl.ANY)] + in_specs
        args = [prev] + args
        aliases = {0: 0}

    def body(*refs):
        i_ref, o_ref = refs[-2], refs[-1]
        o_ref[0] = i_ref[...].T

    return pl.pallas_call(
        body,
        grid=(ns,),
        in_specs=in_specs,
        out_specs=pl.BlockSpec(
            (1, _NH * _C, _R // _NH),
            lambda s: ((q0 + s) // spb, 0, (q0 + s) % spb)),
        out_shape=jax.ShapeDtypeStruct(
            (_BATCH, _NH * _C, _NX * _NZ), jnp.float32),
        input_output_aliases=aliases,
    )(*args)


def kernel(coords, feats):
    n = coords.shape[0]
    pad = jnp.zeros((_NPAD - n, 4), jnp.int32).at[:, 3].set(_BATCH)
    coords_t = jnp.concatenate([coords.astype(jnp.int32), pad], axis=0).T

    tables = []
    r_hbm = None
    for ci, (q0, q1) in enumerate(_CHUNKS):
        call = _make_call(q0, q1, ci == 0)
        if ci == 0:
            table, r_hbm = call(coords_t, feats)
        else:
            table = call(r_hbm, feats)
        tables.append(table)

    if len(_CHUNKS) == 1:
        out = tables[0].reshape(_BATCH, _NX, _NZ, _NH * _C)
        return jnp.transpose(out, (0, 3, 1, 2))
    out = None
    for (q0, q1), table in zip(_CHUNKS, tables):
        out = _tc_transpose_chunk(out, table, q0, q1)
    return out.reshape(_BATCH, _NH * _C, _NX, _NZ)
